# baseline (speedup 1.0000x reference)
import dataclasses
import functools

import jax
import jax.numpy as jnp
from jax import lax
from jax.experimental import pallas as pl
from jax.experimental.pallas import tpu as pltpu
from jax.experimental.pallas import tpu_sc as plsc

_BATCH = 16384
_NIDX = 3 * _BATCH
_NW = 32
_BPW = _NIDX // _NW

_MIN_NORM = 1e-15
_TEMPERATURE = 0.05
_MAX_SCALE = 0.999

_NN = 1000000
_SPLIT_C = 1048576
_SPLIT_G = (_NN + _SPLIT_C - 1) // _SPLIT_C


def _split_body(emb_ref, ids_ref, ox_ref, oy_ref, oi_ref):
    ox_ref[...] = emb_ref[0]
    oy_ref[...] = emb_ref[1]

    @pl.when(pl.program_id(0) == 0)
    def _():
        oi_ref[...] = ids_ref[...].reshape(_NIDX)


def _split_table(embt, idst):
    return pl.pallas_call(
        _split_body,
        grid=(_SPLIT_G,),
        in_specs=[pl.BlockSpec((2, _SPLIT_C), lambda i: (0, i)),
                  pl.BlockSpec((3, _BATCH), lambda i: (0, 0))],
        out_specs=[pl.BlockSpec((_SPLIT_C,), lambda i: (i,)),
                   pl.BlockSpec((_SPLIT_C,), lambda i: (i,)),
                   pl.BlockSpec((_NIDX,), lambda i: (0,))],
        out_shape=[jax.ShapeDtypeStruct((_NN,), jnp.float32),
                   jax.ShapeDtypeStruct((_NN,), jnp.float32),
                   jax.ShapeDtypeStruct((_NIDX,), jnp.int32)],
    )(embt, idst)


def _sc_gather(ext, eyt, ids_flat):
    mesh = plsc.VectorSubcoreMesh(core_axis_name="c", subcore_axis_name="s")
    o1 = jax.ShapeDtypeStruct((_NIDX,), jnp.float32)

    @functools.partial(
        pl.kernel,
        out_type=(o1, o1),
        mesh=mesh,
        scratch_types=[
            pltpu.VMEM((_BPW,), jnp.int32),
            pltpu.VMEM((_BPW,), jnp.float32),
            pltpu.VMEM((_BPW,), jnp.float32),
            pltpu.SemaphoreType.DMA,
            pltpu.SemaphoreType.DMA,
            pltpu.SemaphoreType.DMA,
            pltpu.SemaphoreType.DMA,
        ],
    )
    def gather_k(ext_hbm, eyt_hbm, ids_hbm, outx_hbm, outy_hbm,
                 idx_v, rx_v, ry_v, semi, semx, semy, semw):
        wid = lax.axis_index("s") * 2 + lax.axis_index("c")
        base = wid * _BPW
        h = _BPW // 2
        s0, s1 = pl.ds(base, h), pl.ds(base + h, h)
        l0, l1 = pl.ds(0, h), pl.ds(h, h)
        i0 = pltpu.async_copy(ids_hbm.at[s0], idx_v.at[l0], semi)
        i1 = pltpu.async_copy(ids_hbm.at[s1], idx_v.at[l1], semi)
        i0.wait()
        gx0 = pltpu.async_copy(ext_hbm.at[idx_v.at[l0]], rx_v.at[l0], semx)
        gy0 = pltpu.async_copy(eyt_hbm.at[idx_v.at[l0]], ry_v.at[l0], semy)
        i1.wait()
        gx1 = pltpu.async_copy(ext_hbm.at[idx_v.at[l1]], rx_v.at[l1], semx)
        gy1 = pltpu.async_copy(eyt_hbm.at[idx_v.at[l1]], ry_v.at[l1], semy)
        gx0.wait()
        wx0 = pltpu.async_copy(rx_v.at[l0], outx_hbm.at[s0], semw)
        gy0.wait()
        wy0 = pltpu.async_copy(ry_v.at[l0], outy_hbm.at[s0], semw)
        gx1.wait()
        wx1 = pltpu.async_copy(rx_v.at[l1], outx_hbm.at[s1], semw)
        gy1.wait()
        wy1 = pltpu.async_copy(ry_v.at[l1], outy_hbm.at[s1], semw)
        wx0.wait()
        wy0.wait()
        wx1.wait()
        wy1.wait()

    return gather_k(ext, eyt, ids_flat)


def _lca_dist(ax, ay, bx, by):
    a2 = ax * ax + ay * ay
    rx = ax / a2
    ry = ay / a2
    r2 = rx * rx + ry * ry - 1.0
    ux = bx - rx
    uy = by - ry
    u2 = ux * ux + uy * uy
    f = r2 / u2
    yix = f * ux + rx
    yiy = f * uy + ry
    xta = ax * yix + ay * yiy
    na = jnp.maximum(yix * yix + yiy * yiy, _MIN_NORM)
    g = 2.0 * xta / na
    ox = g * yix - ax
    oy = g * yiy - ay
    vx = ox - rx
    vy = oy - ry
    v2 = vx * vx + vy * vy
    h = r2 / v2
    wx = h * vx + rx
    wy = h * vy + ry
    w2 = wx * wx + wy * wy
    denom = 1.0 + jnp.sqrt(1.0 - w2)
    px = wx / denom
    py = wy / denom
    pn = jnp.sqrt(px * px + py * py)
    return jnp.log((1.0 + pn) / (1.0 - pn))


def _tc_body(scale_ref, ex_ref, ey_ref, sim_ref, o_ref):
    s = jnp.clip(scale_ref[0, 0], 0.01, _MAX_SCALE)

    def norm_xy(i):
        x = ex_ref[i]
        y = ey_ref[i]
        n = jnp.maximum(jnp.sqrt(x * x + y * y), 1e-12)
        fac = s / n
        return x * fac, y * fac

    e1x, e1y = norm_xy(0)
    e2x, e2y = norm_xy(1)
    e3x, e3y = norm_xy(2)

    d12 = _lca_dist(e1x, e1y, e2x, e2y)
    d13 = _lca_dist(e1x, e1y, e3x, e3y)
    d23 = _lca_dist(e2x, e2y, e3x, e3y)

    inv_t = 1.0 / _TEMPERATURE
    z1 = d12 * inv_t
    z2 = d13 * inv_t
    z3 = d23 * inv_t
    m = jnp.maximum(jnp.maximum(z1, z2), z3)
    q1 = jnp.exp(z1 - m)
    q2 = jnp.exp(z2 - m)
    q3 = jnp.exp(z3 - m)
    qs = q1 + q2 + q3

    s1 = sim_ref[0]
    s2 = sim_ref[1]
    s3 = sim_ref[2]
    w_ord = (s1 * q1 + s2 * q2 + s3 * q3) / qs
    total = (s1 + s2 + s3) - w_ord
    o_ref[0, 0] = jnp.sum(total) * (1.0 / _BATCH)


def _tc_loss(scale, ex3, ey3, sim3):
    return pl.pallas_call(
        _tc_body,
        out_shape=jax.ShapeDtypeStruct((1, 1), jnp.float32),
        in_specs=[pl.BlockSpec(memory_space=pltpu.SMEM)]
        + [pl.BlockSpec(memory_space=pltpu.VMEM)] * 3,
        out_specs=pl.BlockSpec(memory_space=pltpu.SMEM),
    )(scale, ex3, ey3, sim3)


def kernel(triple_ids, similarities, embeddings, scale):
    ext, eyt, ids_flat = _split_table(embeddings.T, triple_ids.T)
    ex, ey = _sc_gather(ext, eyt, ids_flat)
    ex3 = ex.reshape(3, 128, 128)
    ey3 = ey.reshape(3, 128, 128)
    sim3 = similarities.T.reshape(3, 128, 128)
    out = _tc_loss(scale.reshape(1, 1), ex3, ey3, sim3)
    return out[0, 0]


# speedup vs baseline: 1.0259x; 1.0259x over previous
import functools

import jax
import jax.numpy as jnp
from jax import lax
from jax.experimental import pallas as pl
from jax.experimental.pallas import tpu as pltpu
from jax.experimental.pallas import tpu_sc as plsc

_BATCH = 16384
_NIDX = 3 * _BATCH
_NW = 32
_BPW = _NIDX // _NW

_MIN_NORM = 1e-15
_TEMPERATURE = 0.05
_MAX_SCALE = 0.999

_NN = 1000000
_SPLIT_C = 524288
_SPLIT_G = 2


def _split_body(emb_ref, ids_ref, ox_ref, oy_ref, oi_ref):
    ox_ref[...] = emb_ref[0]
    oy_ref[...] = emb_ref[1]

    @pl.when(pl.program_id(0) == 0)
    def _():
        oi_ref[...] = ids_ref[...].reshape(_NIDX)


def _split_table(embt, idst):
    return pl.pallas_call(
        _split_body,
        grid=(_SPLIT_G,),
        in_specs=[pl.BlockSpec((2, _SPLIT_C), lambda i: (0, i)),
                  pl.BlockSpec((3, _BATCH), lambda i: (0, 0))],
        out_specs=[pl.BlockSpec((_SPLIT_C,), lambda i: (i,)),
                   pl.BlockSpec((_SPLIT_C,), lambda i: (i,)),
                   pl.BlockSpec((_NIDX,), lambda i: (0,))],
        out_shape=[jax.ShapeDtypeStruct((_NN,), jnp.float32),
                   jax.ShapeDtypeStruct((_NN,), jnp.float32),
                   jax.ShapeDtypeStruct((_NIDX,), jnp.int32)],
    )(embt, idst)


def _sc_gather(ext, eyt, ids_flat):
    mesh = plsc.VectorSubcoreMesh(core_axis_name="c", subcore_axis_name="s")
    o1 = jax.ShapeDtypeStruct((_NIDX,), jnp.float32)

    @functools.partial(
        pl.kernel,
        out_type=(o1, o1),
        mesh=mesh,
        scratch_types=[
            pltpu.VMEM((_BPW,), jnp.int32),
            pltpu.VMEM((_BPW,), jnp.float32),
            pltpu.VMEM((_BPW,), jnp.float32),
            pltpu.SemaphoreType.DMA,
            pltpu.SemaphoreType.DMA,
            pltpu.SemaphoreType.DMA,
            pltpu.SemaphoreType.DMA,
        ],
    )
    def gather_k(ext_hbm, eyt_hbm, ids_hbm, outx_hbm, outy_hbm,
                 idx_v, rx_v, ry_v, semi, semx, semy, semw):
        wid = lax.axis_index("s") * 2 + lax.axis_index("c")
        base = wid * _BPW
        h = _BPW // 2
        s0, s1 = pl.ds(base, h), pl.ds(base + h, h)
        l0, l1 = pl.ds(0, h), pl.ds(h, h)
        i0 = pltpu.async_copy(ids_hbm.at[s0], idx_v.at[l0], semi)
        i1 = pltpu.async_copy(ids_hbm.at[s1], idx_v.at[l1], semi)
        i0.wait()
        gx0 = pltpu.async_copy(ext_hbm.at[idx_v.at[l0]], rx_v.at[l0], semx)
        gy0 = pltpu.async_copy(eyt_hbm.at[idx_v.at[l0]], ry_v.at[l0], semy)
        i1.wait()
        gx1 = pltpu.async_copy(ext_hbm.at[idx_v.at[l1]], rx_v.at[l1], semx)
        gy1 = pltpu.async_copy(eyt_hbm.at[idx_v.at[l1]], ry_v.at[l1], semy)
        gx0.wait()
        wx0 = pltpu.async_copy(rx_v.at[l0], outx_hbm.at[s0], semw)
        gy0.wait()
        wy0 = pltpu.async_copy(ry_v.at[l0], outy_hbm.at[s0], semw)
        gx1.wait()
        wx1 = pltpu.async_copy(rx_v.at[l1], outx_hbm.at[s1], semw)
        gy1.wait()
        wy1 = pltpu.async_copy(ry_v.at[l1], outy_hbm.at[s1], semw)
        wx0.wait()
        wy0.wait()
        wx1.wait()
        wy1.wait()

    return gather_k(ext, eyt, ids_flat)


def _lca_dist(ax, ay, bx, by):
    a2 = ax * ax + ay * ay
    rx = ax / a2
    ry = ay / a2
    r2 = rx * rx + ry * ry - 1.0
    ux = bx - rx
    uy = by - ry
    u2 = ux * ux + uy * uy
    f = r2 / u2
    yix = f * ux + rx
    yiy = f * uy + ry
    xta = ax * yix + ay * yiy
    na = jnp.maximum(yix * yix + yiy * yiy, _MIN_NORM)
    g = 2.0 * xta / na
    ox = g * yix - ax
    oy = g * yiy - ay
    vx = ox - rx
    vy = oy - ry
    v2 = vx * vx + vy * vy
    h = r2 / v2
    wx = h * vx + rx
    wy = h * vy + ry
    w2 = wx * wx + wy * wy
    denom = 1.0 + jnp.sqrt(1.0 - w2)
    px = wx / denom
    py = wy / denom
    pn = jnp.sqrt(px * px + py * py)
    return jnp.log((1.0 + pn) / (1.0 - pn))


def _tc_body(scale_ref, ex_ref, ey_ref, sim_ref, o_ref):
    s = jnp.clip(scale_ref[0, 0], 0.01, _MAX_SCALE)

    def norm_xy(i):
        x = ex_ref[i]
        y = ey_ref[i]
        n = jnp.maximum(jnp.sqrt(x * x + y * y), 1e-12)
        fac = s / n
        return x * fac, y * fac

    e1x, e1y = norm_xy(0)
    e2x, e2y = norm_xy(1)
    e3x, e3y = norm_xy(2)

    d12 = _lca_dist(e1x, e1y, e2x, e2y)
    d13 = _lca_dist(e1x, e1y, e3x, e3y)
    d23 = _lca_dist(e2x, e2y, e3x, e3y)

    inv_t = 1.0 / _TEMPERATURE
    z1 = d12 * inv_t
    z2 = d13 * inv_t
    z3 = d23 * inv_t
    m = jnp.maximum(jnp.maximum(z1, z2), z3)
    q1 = jnp.exp(z1 - m)
    q2 = jnp.exp(z2 - m)
    q3 = jnp.exp(z3 - m)
    qs = q1 + q2 + q3

    s1 = sim_ref[0]
    s2 = sim_ref[1]
    s3 = sim_ref[2]
    w_ord = (s1 * q1 + s2 * q2 + s3 * q3) / qs
    total = (s1 + s2 + s3) - w_ord
    o_ref[0, 0] = jnp.sum(total) * (1.0 / _BATCH)


def _tc_loss(scale, ex3, ey3, sim3):
    return pl.pallas_call(
        _tc_body,
        out_shape=jax.ShapeDtypeStruct((1, 1), jnp.float32),
        in_specs=[pl.BlockSpec(memory_space=pltpu.SMEM)]
        + [pl.BlockSpec(memory_space=pltpu.VMEM)] * 3,
        out_specs=pl.BlockSpec(memory_space=pltpu.SMEM),
    )(scale, ex3, ey3, sim3)


def kernel(triple_ids, similarities, embeddings, scale):
    ext, eyt, ids_flat = _split_table(embeddings.T, triple_ids.T)
    ex, ey = _sc_gather(ext, eyt, ids_flat)
    ex3 = ex.reshape(3, 128, 128)
    ey3 = ey.reshape(3, 128, 128)
    sim3 = similarities.T.reshape(3, 128, 128)
    out = _tc_loss(scale.reshape(1, 1), ex3, ey3, sim3)
    return out[0, 0]
